# baseline (speedup 1.0000x reference)
import functools
import math

import jax
import jax.numpy as jnp
from jax.experimental import pallas as pl

_F32 = jnp.float32


def _fps_body(xyzt_ref, out_ref, *, npoint, n):
    x = xyzt_ref[0]
    y = xyzt_ref[1]
    z = xyzt_ref[2]
    b = x.shape[0]
    iota = jax.lax.broadcasted_iota(jnp.int32, (b, n), 1).astype(_F32)
    big = jnp.float32(n)

    def body(i, state):
        dist, far = state
        onehot = (iota == far).astype(_F32)
        cx = jnp.sum(onehot * x, axis=1, keepdims=True)
        cy = jnp.sum(onehot * y, axis=1, keepdims=True)
        cz = jnp.sum(onehot * z, axis=1, keepdims=True)
        out_ref[pl.ds(i, 1)] = jnp.concatenate(
            [cx[None], cy[None], cz[None]], axis=-1)
        dx = x - cx
        dy = y - cy
        dz = z - cz
        d = (dx * dx + dy * dy) + dz * dz
        dist = jnp.minimum(dist, d)
        m = jnp.max(dist, axis=1, keepdims=True)
        far = jnp.min(jnp.where(dist == m, iota, big), axis=1, keepdims=True)
        return dist, far

    dist0 = jnp.full((b, n), 1e10, _F32)
    far0 = jnp.zeros((b, 1), _F32)
    jax.lax.fori_loop(0, npoint, body, (dist0, far0))


def _fps(xyzt, npoint):
    _, b, n = xyzt.shape
    out = pl.pallas_call(
        functools.partial(_fps_body, npoint=npoint, n=n),
        out_shape=jax.ShapeDtypeStruct((npoint, b, 3), _F32),
    )(xyzt)
    return jnp.transpose(out, (1, 0, 2))


def _sa_body(xyzt_ref, val_ref, nx_ref, nxpad_ref,
             w1_ref, b1_ref, w2_ref, b2_ref, w3_ref, b3_ref, out_ref,
             *, r2, k, n, s, s_tile, c_in):
    xyzt = xyzt_ref[0]
    val = val_ref[0]
    val1 = jnp.dot(val, w1_ref[...], preferred_element_type=_F32)
    nxw = jnp.dot(nxpad_ref[0], w1_ref[...],
                  preferred_element_type=_F32)
    p2 = jnp.sum(xyzt * xyzt, axis=0, keepdims=True)
    c1 = val1.shape[-1]

    for t in range(s // s_tile):
        nx = nx_ref[0, t * s_tile:(t + 1) * s_tile, :]
        cross = jnp.dot(nx, xyzt, preferred_element_type=_F32)
        s2 = jnp.sum(nx * nx, axis=1, keepdims=True)
        sq = s2 + p2 - 2.0 * cross
        maskf = (sq <= r2).astype(_F32)

        jio = jax.lax.broadcasted_iota(jnp.int32, (s_tile, n), 1)
        cum = maskf
        sh = 1
        while sh < n:
            cum = cum + jnp.where(jio >= sh, jnp.roll(cum, sh, axis=1), 0.0)
            sh *= 2
        posm = jnp.where(maskf > 0, cum - maskf, -1.0)
        kio = jax.lax.broadcasted_iota(
            jnp.int32, (s_tile, k, n), 1).astype(_F32)
        oh = (posm[:, None, :] == kio).astype(_F32)
        oh2 = oh.reshape(s_tile * k, n)
        g = jnp.dot(oh2, val1, preferred_element_type=_F32)
        rowsum = jnp.sum(oh2, axis=1, keepdims=True)
        first = (posm == 0.0).astype(_F32)
        gfirst = jnp.dot(first, val1, preferred_element_type=_F32)
        gf = jnp.broadcast_to(gfirst[:, None, :],
                              (s_tile, k, c1)).reshape(s_tile * k, c1)
        nxb = jnp.broadcast_to(
            nxw[t * s_tile:(t + 1) * s_tile][:, None, :],
            (s_tile, k, c1)).reshape(s_tile * k, c1)
        x = g + (1.0 - rowsum) * gf - nxb

        x = jnp.maximum(x + b1_ref[...], 0.0)
        x = jnp.maximum(jnp.dot(x, w2_ref[...], preferred_element_type=_F32)
                        + b2_ref[...], 0.0)
        x = jnp.maximum(jnp.dot(x, w3_ref[...], preferred_element_type=_F32)
                        + b3_ref[...], 0.0)
        c_out = x.shape[-1]
        out_ref[0, t * s_tile:(t + 1) * s_tile, :] = jnp.max(
            x.reshape(s_tile, k, c_out), axis=1)


def _sa(xyzt, val, nx, nxpad, ws, radius, k, s_tile):
    n = xyzt.shape[2]
    bsz, s = nx.shape[0], nx.shape[1]
    c_in = val.shape[2]
    c_out = ws[2][0].shape[1]
    body = functools.partial(_sa_body, r2=radius * radius, k=k, n=n,
                             s=s, s_tile=s_tile, c_in=c_in)
    w_specs = []
    for w, b in ws:
        w_specs += [pl.BlockSpec(w.shape, lambda i: (0, 0)),
                    pl.BlockSpec(b.shape, lambda i: (0, 0))]
    out = pl.pallas_call(
        body,
        grid=(bsz,),
        in_specs=[
            pl.BlockSpec((1, 3, n), lambda i: (i, 0, 0)),
            pl.BlockSpec((1, n, c_in), lambda i: (i, 0, 0)),
            pl.BlockSpec((1, s, 3), lambda i: (i, 0, 0)),
            pl.BlockSpec((1, s, c_in), lambda i: (i, 0, 0)),
            *w_specs,
        ],
        out_specs=pl.BlockSpec((1, s, c_out), lambda i: (i, 0, 0)),
        out_shape=jax.ShapeDtypeStruct((bsz, s, c_out), _F32),
    )(xyzt, val, nx, nxpad,
      ws[0][0], ws[0][1], ws[1][0], ws[1][1], ws[2][0], ws[2][1])
    return out


def _head_body(x_ref, w1_ref, b1_ref, w2_ref, b2_ref, w3_ref, b3_ref,
               f1w_ref, f1b_ref, f2w_ref, f2b_ref, f3w_ref, f3b_ref,
               out_ref, *, bsz, npts):
    x = x_ref[...]
    x = jnp.maximum(jnp.dot(x, w1_ref[...], preferred_element_type=_F32)
                    + b1_ref[...], 0.0)
    x = jnp.maximum(jnp.dot(x, w2_ref[...], preferred_element_type=_F32)
                    + b2_ref[...], 0.0)
    x = jnp.maximum(jnp.dot(x, w3_ref[...], preferred_element_type=_F32)
                    + b3_ref[...], 0.0)
    x = jnp.max(x.reshape(bsz, npts, x.shape[-1]), axis=1)
    x = jnp.maximum(jnp.dot(x, f1w_ref[...], preferred_element_type=_F32)
                    + f1b_ref[...], 0.0)
    x = jnp.maximum(jnp.dot(x, f2w_ref[...], preferred_element_type=_F32)
                    + f2b_ref[...], 0.0)
    out_ref[...] = (jnp.dot(x, f3w_ref[...], preferred_element_type=_F32)
                    + f3b_ref[...])


def _head(x, ws, fcs, bsz, npts):
    flat = [a for pair in (list(ws) + list(fcs)) for a in pair]
    return pl.pallas_call(
        functools.partial(_head_body, bsz=bsz, npts=npts),
        out_shape=jax.ShapeDtypeStruct((bsz, fcs[2][0].shape[1]), _F32),
    )(x.reshape(bsz * npts, x.shape[-1]), *flat)


def _fold_bn(p):
    s = 1.0 / math.sqrt(1.0 + 1e-5)
    g = p["g"] * s
    return p["W"] * g[None, :], (p["b"] * g + p["be"])[None, :]


def kernel(xyz, params):
    bsz = xyz.shape[0]
    xyzt = jnp.transpose(xyz, (1, 0, 2))
    xyz_bn3 = jnp.transpose(xyz, (0, 2, 1))

    sa1 = [_fold_bn(p) for p in params["sa1"]]
    sa2 = [_fold_bn(p) for p in params["sa2"]]
    sa3 = [_fold_bn(p) for p in params["sa3"]]
    fc1 = _fold_bn(params["fc1"])
    fc2 = _fold_bn(params["fc2"])
    fc3 = (params["fc3"]["W"], params["fc3"]["b"][None, :])

    nx1 = _fps(xyzt, 512)
    l1p = _sa(xyz, xyz_bn3, nx1, nx1, sa1, 0.2, 32, 64)

    nx1t = jnp.transpose(nx1, (0, 2, 1))
    nx2 = _fps(jnp.transpose(nx1t, (1, 0, 2)), 128)
    val2 = jnp.concatenate([nx1, l1p], axis=-1)
    nx2pad = jnp.concatenate(
        [nx2, jnp.zeros((bsz, 128, val2.shape[-1] - 3), _F32)], axis=-1)
    l2p = _sa(nx1t, val2, nx2, nx2pad, sa2, 0.4, 64, 64)

    x3 = jnp.concatenate([nx2, l2p], axis=-1)
    return _head(x3, sa3, (fc1, fc2, fc3), bsz, 128)


# speedup vs baseline: 1.0877x; 1.0877x over previous
import functools
import math

import jax
import jax.numpy as jnp
from jax.experimental import pallas as pl

_F32 = jnp.float32


def _fps_body(xyzt_ref, out_ref, *, npoint, n):
    x = xyzt_ref[0]
    y = xyzt_ref[1]
    z = xyzt_ref[2]
    b = x.shape[0]
    iota = jax.lax.broadcasted_iota(jnp.int32, (b, n), 1).astype(_F32)
    big = jnp.float32(n)

    def body(i, state):
        dist, far = state
        onehot = (iota == far).astype(_F32)
        cx = jnp.sum(onehot * x, axis=1, keepdims=True)
        cy = jnp.sum(onehot * y, axis=1, keepdims=True)
        cz = jnp.sum(onehot * z, axis=1, keepdims=True)
        out_ref[pl.ds(i, 1)] = jnp.concatenate(
            [cx[None], cy[None], cz[None]], axis=-1)
        dx = x - cx
        dy = y - cy
        dz = z - cz
        d = (dx * dx + dy * dy) + dz * dz
        dist = jnp.minimum(dist, d)
        m = jnp.max(dist, axis=1, keepdims=True)
        far = jnp.min(jnp.where(dist == m, iota, big), axis=1, keepdims=True)
        return dist, far

    dist0 = jnp.full((b, n), 1e10, _F32)
    far0 = jnp.zeros((b, 1), _F32)
    jax.lax.fori_loop(0, npoint, body, (dist0, far0))


def _fps(xyzt, npoint):
    _, b, n = xyzt.shape
    out = pl.pallas_call(
        functools.partial(_fps_body, npoint=npoint, n=n),
        out_shape=jax.ShapeDtypeStruct((npoint, b, 3), _F32),
    )(xyzt)
    return jnp.transpose(out, (1, 0, 2))


def _sa_body(xyzt_ref, val_ref, nx_ref, nxpad_ref,
             w1_ref, b1_ref, w2_ref, b2_ref, w3_ref, b3_ref, out_ref,
             *, r2, k, n, s, s_tile, c_in):
    xyzt = xyzt_ref[0]
    val = val_ref[0]
    val1 = jnp.dot(val, w1_ref[...], preferred_element_type=_F32)
    nxw = jnp.dot(nxpad_ref[0], w1_ref[...],
                  preferred_element_type=_F32)
    p2 = jnp.sum(xyzt * xyzt, axis=0, keepdims=True)
    c1 = val1.shape[-1]
    val1b = val1.astype(jnp.bfloat16)
    kio = jax.lax.broadcasted_iota(jnp.int32, (s_tile, k, n), 1).astype(_F32)
    kflat = jax.lax.broadcasted_iota(
        jnp.int32, (s_tile, k, 1), 1).astype(_F32).reshape(s_tile * k, 1)

    for t in range(s // s_tile):
        nx = nx_ref[0, t * s_tile:(t + 1) * s_tile, :]
        cross = jnp.dot(nx, xyzt, preferred_element_type=_F32)
        s2 = jnp.sum(nx * nx, axis=1, keepdims=True)
        sq = s2 + p2 - 2.0 * cross
        maskf = (sq <= r2).astype(_F32)

        jio = jax.lax.broadcasted_iota(jnp.int32, (s_tile, n), 1)
        cum = maskf
        sh = 1
        while sh < n:
            cum = cum + jnp.where(jio >= sh, jnp.roll(cum, sh, axis=1), 0.0)
            sh *= 2
        posm = jnp.where(maskf > 0, cum - maskf, -1.0)
        oh = (posm[:, None, :] == kio).astype(jnp.bfloat16)
        oh2 = oh.reshape(s_tile * k, n)
        g = jnp.dot(oh2, val1b, preferred_element_type=_F32)
        count = jnp.sum(maskf, axis=1, keepdims=True)
        countf = jnp.broadcast_to(
            count[:, None, :], (s_tile, k, 1)).reshape(s_tile * k, 1)
        empty = (kflat >= countf).astype(_F32)
        first = (posm == 0.0).astype(_F32)
        gfirst = jnp.dot(first, val1, preferred_element_type=_F32)
        gf = jnp.broadcast_to(gfirst[:, None, :],
                              (s_tile, k, c1)).reshape(s_tile * k, c1)
        nxb = jnp.broadcast_to(
            nxw[t * s_tile:(t + 1) * s_tile][:, None, :],
            (s_tile, k, c1)).reshape(s_tile * k, c1)
        x = g + empty * gf - nxb

        x = jnp.maximum(x + b1_ref[...], 0.0)
        x = jnp.maximum(jnp.dot(x, w2_ref[...], preferred_element_type=_F32)
                        + b2_ref[...], 0.0)
        x = jnp.maximum(jnp.dot(x, w3_ref[...], preferred_element_type=_F32)
                        + b3_ref[...], 0.0)
        c_out = x.shape[-1]
        out_ref[0, t * s_tile:(t + 1) * s_tile, :] = jnp.max(
            x.reshape(s_tile, k, c_out), axis=1)


def _sa(xyzt, val, nx, nxpad, ws, radius, k, s_tile):
    n = xyzt.shape[2]
    bsz, s = nx.shape[0], nx.shape[1]
    c_in = val.shape[2]
    c_out = ws[2][0].shape[1]
    body = functools.partial(_sa_body, r2=radius * radius, k=k, n=n,
                             s=s, s_tile=s_tile, c_in=c_in)
    w_specs = []
    for w, b in ws:
        w_specs += [pl.BlockSpec(w.shape, lambda i: (0, 0)),
                    pl.BlockSpec(b.shape, lambda i: (0, 0))]
    out = pl.pallas_call(
        body,
        grid=(bsz,),
        in_specs=[
            pl.BlockSpec((1, 3, n), lambda i: (i, 0, 0)),
            pl.BlockSpec((1, n, c_in), lambda i: (i, 0, 0)),
            pl.BlockSpec((1, s, 3), lambda i: (i, 0, 0)),
            pl.BlockSpec((1, s, c_in), lambda i: (i, 0, 0)),
            *w_specs,
        ],
        out_specs=pl.BlockSpec((1, s, c_out), lambda i: (i, 0, 0)),
        out_shape=jax.ShapeDtypeStruct((bsz, s, c_out), _F32),
    )(xyzt, val, nx, nxpad,
      ws[0][0], ws[0][1], ws[1][0], ws[1][1], ws[2][0], ws[2][1])
    return out


def _head_body(x_ref, w1_ref, b1_ref, w2_ref, b2_ref, w3_ref, b3_ref,
               f1w_ref, f1b_ref, f2w_ref, f2b_ref, f3w_ref, f3b_ref,
               out_ref, *, bsz, npts):
    x = x_ref[...]
    x = jnp.maximum(jnp.dot(x, w1_ref[...], preferred_element_type=_F32)
                    + b1_ref[...], 0.0)
    x = jnp.maximum(jnp.dot(x, w2_ref[...], preferred_element_type=_F32)
                    + b2_ref[...], 0.0)
    x = jnp.maximum(jnp.dot(x, w3_ref[...], preferred_element_type=_F32)
                    + b3_ref[...], 0.0)
    x = jnp.max(x.reshape(bsz, npts, x.shape[-1]), axis=1)
    x = jnp.maximum(jnp.dot(x, f1w_ref[...], preferred_element_type=_F32)
                    + f1b_ref[...], 0.0)
    x = jnp.maximum(jnp.dot(x, f2w_ref[...], preferred_element_type=_F32)
                    + f2b_ref[...], 0.0)
    out_ref[...] = (jnp.dot(x, f3w_ref[...], preferred_element_type=_F32)
                    + f3b_ref[...])


def _head(x, ws, fcs, bsz, npts):
    flat = [a for pair in (list(ws) + list(fcs)) for a in pair]
    return pl.pallas_call(
        functools.partial(_head_body, bsz=bsz, npts=npts),
        out_shape=jax.ShapeDtypeStruct((bsz, fcs[2][0].shape[1]), _F32),
    )(x.reshape(bsz * npts, x.shape[-1]), *flat)


def _fold_bn(p):
    s = 1.0 / math.sqrt(1.0 + 1e-5)
    g = p["g"] * s
    return p["W"] * g[None, :], (p["b"] * g + p["be"])[None, :]


def kernel(xyz, params):
    bsz = xyz.shape[0]
    xyzt = jnp.transpose(xyz, (1, 0, 2))
    xyz_bn3 = jnp.transpose(xyz, (0, 2, 1))

    sa1 = [_fold_bn(p) for p in params["sa1"]]
    sa2 = [_fold_bn(p) for p in params["sa2"]]
    sa3 = [_fold_bn(p) for p in params["sa3"]]
    fc1 = _fold_bn(params["fc1"])
    fc2 = _fold_bn(params["fc2"])
    fc3 = (params["fc3"]["W"], params["fc3"]["b"][None, :])

    nx1 = _fps(xyzt, 512)
    l1p = _sa(xyz, xyz_bn3, nx1, nx1, sa1, 0.2, 32, 64)

    nx1t = jnp.transpose(nx1, (0, 2, 1))
    nx2 = _fps(jnp.transpose(nx1t, (1, 0, 2)), 128)
    val2 = jnp.concatenate([nx1, l1p], axis=-1)
    nx2pad = jnp.concatenate(
        [nx2, jnp.zeros((bsz, 128, val2.shape[-1] - 3), _F32)], axis=-1)
    l2p = _sa(nx1t, val2, nx2, nx2pad, sa2, 0.4, 64, 64)

    x3 = jnp.concatenate([nx2, l2p], axis=-1)
    return _head(x3, sa3, (fc1, fc2, fc3), bsz, 128)
